# Initial kernel scaffold; baseline (speedup 1.0000x reference)
import jax
import jax.numpy as jnp
from jax.experimental import pallas as pl


def kernel(x, W0, b0, W1, b1, Wfc, bfc, D0, D1, spirals0, spirals1):
    raise NotImplementedError("write your pallas kernel here")


# speedup vs baseline: 14.9520x; 14.9520x over previous
import jax
import jax.numpy as jnp
from jax import lax
from jax.experimental import pallas as pl
from jax.experimental.pallas import tpu as pltpu
from jax.experimental.pallas import tpu_sc as plsc


def _sc_gather(table, idx2d, n_rows, d, c_per_w, c_stride, dpad):
    info = plsc.get_sparse_core_info()
    NC, NS = info.num_cores, info.num_subcores
    groups, rem = divmod(c_per_w, 4)
    mesh = plsc.VectorSubcoreMesh(core_axis_name="c", subcore_axis_name="s")

    def body(table_ref, idx_ref, out_ref, idx_v, buf_v, sem):
        w = lax.axis_index("s") * NC + lax.axis_index("c")
        cbase = w * c_per_w
        pltpu.sync_copy(idx_ref.at[pl.ds(w * c_stride, c_stride)], idx_v)

        def _src(buf):
            return buf if d == dpad else buf.at[:, pl.ds(0, d)]

        def grp(j, carry):
            handles = [
                pltpu.async_copy(
                    table_ref.at[idx_v.at[4 * j + t]],
                    buf_v.at[pl.ds(t * 128, 128)], sem)
                for t in range(4)
            ]
            for h in handles:
                h.wait()
            pltpu.sync_copy(_src(buf_v),
                            out_ref.at[pl.ds((cbase + 4 * j) * 128, 512)])
            return carry

        lax.fori_loop(0, groups, grp, 0)
        if rem:
            hs = [
                pltpu.async_copy(
                    table_ref.at[idx_v.at[4 * groups + t]],
                    buf_v.at[pl.ds(t * 128, 128)], sem)
                for t in range(rem)
            ]
            for h in hs:
                h.wait()
            pltpu.sync_copy(
                _src(buf_v.at[pl.ds(0, rem * 128)]),
                out_ref.at[pl.ds((cbase + 4 * groups) * 128, rem * 128)])

    fn = pl.kernel(
        body,
        mesh=mesh,
        out_type=jax.ShapeDtypeStruct((n_rows, d), jnp.float32),
        scratch_types=[
            pltpu.VMEM((c_stride, 128), jnp.int32),
            pltpu.VMEM((512, dpad), jnp.float32),
            pltpu.SemaphoreType.DMA,
        ],
        compiler_params=pltpu.CompilerParams(use_tc_tiling_on_sc=False),
    )
    return fn(table, idx2d)


def _spiral_mm(g, Wr, b2, n_v, v_blk, n_vblk, s, f_in, f_out):
    def body(g_ref, w_ref, b_ref, o_ref):
        j = pl.program_id(1)
        acc = jnp.dot(g_ref[0, 0], w_ref[0],
                      preferred_element_type=jnp.float32)
        for si in range(1, s):
            acc += jnp.dot(g_ref[si, 0], w_ref[si],
                           preferred_element_type=jnp.float32)
        acc += b_ref[0]
        acc = jnp.where(acc > 0, acc, jnp.exp(acc) - 1.0)
        v = j * v_blk + lax.broadcasted_iota(jnp.int32, (v_blk, 1), 0)
        acc = jnp.where(v == n_v - 1, 0.0, acc)
        o_ref[0] = acc

    return pl.pallas_call(
        body,
        grid=(8, n_vblk),
        in_specs=[
            pl.BlockSpec((s, 1, v_blk, f_in), lambda b, j: (0, b, j, 0)),
            pl.BlockSpec((s, f_in, f_out), lambda b, j: (0, 0, 0)),
            pl.BlockSpec((1, f_out), lambda b, j: (0, 0)),
        ],
        out_specs=pl.BlockSpec((1, v_blk, f_out), lambda b, j: (b, j, 0)),
        out_shape=jax.ShapeDtypeStruct((8, n_v, f_out), jnp.float32),
    )(g, Wr, b2)


def _down_mm(D, h, r_blk, n_rblk, f):
    R, V = D.shape

    def body(d_ref, h_ref, o_ref):
        for b in range(8):
            o_ref[b] = jnp.dot(d_ref[...], h_ref[b],
                               preferred_element_type=jnp.float32)

    return pl.pallas_call(
        body,
        grid=(n_rblk,),
        in_specs=[
            pl.BlockSpec((r_blk, V), lambda r: (r, 0)),
            pl.BlockSpec((8, V, f), lambda r: (0, 0, 0)),
        ],
        out_specs=pl.BlockSpec((8, r_blk, f), lambda r: (0, r, 0)),
        out_shape=jax.ShapeDtypeStruct((8, R, f), jnp.float32),
    )(D, h)


def _fc(h3f, Wfc, bfc2, k_blk, n_kblk, latent):
    def body(h_ref, w_ref, b_ref, o_ref):
        @pl.when(pl.program_id(0) == 0)
        def _():
            o_ref[...] = jnp.broadcast_to(b_ref[0], (8, latent))

        o_ref[...] += jnp.dot(h_ref[...], w_ref[...],
                              preferred_element_type=jnp.float32)

    return pl.pallas_call(
        body,
        grid=(n_kblk,),
        in_specs=[
            pl.BlockSpec((8, k_blk), lambda k: (0, k)),
            pl.BlockSpec((k_blk, latent), lambda k: (k, 0)),
            pl.BlockSpec((1, latent), lambda k: (0, 0)),
        ],
        out_specs=pl.BlockSpec((8, latent), lambda k: (0, 0)),
        out_shape=jax.ShapeDtypeStruct((8, latent), jnp.float32),
    )(h3f, Wfc, bfc2)


def kernel(x, W0, b0, W1, b1, Wfc, bfc, D0, D1, spirals0, spirals1):
    B = 8
    N0, N1, N2 = 5024, 1257, 315
    S = 16
    F0, F1, F2, LAT = 64, 128, 256, 256

    offs0 = (jnp.arange(B, dtype=jnp.int32) * N0)
    idx0 = spirals0.T[:, None, :] + offs0[None, :, None]
    NR0 = S * B * N0
    idx0_2d = jnp.pad(idx0.reshape(32, 157, 128),
                      ((0, 0), (0, 3), (0, 0))).reshape(32 * 160, 128)
    table0 = x.reshape(B * N0, F0)
    g0 = _sc_gather(table0, idx0_2d, NR0, F0, c_per_w=157, c_stride=160,
                    dpad=F0)
    g0 = g0.reshape(S, B, N0, F0)

    W0r = W0.reshape(S, F0, F1)
    h0 = _spiral_mm(g0, W0r, b0.reshape(1, F1), N0, 1256, 4, S, F0, F1)
    h1 = _down_mm(D0, h0, 128, 10, F1)

    V1P = 1280
    offs1 = (jnp.arange(B, dtype=jnp.int32) * N1)
    idx1 = spirals1.T[:, None, :] + offs1[None, :, None]
    idx1 = jnp.pad(idx1, ((0, 0), (0, 0), (0, V1P - N1)))
    NR1 = S * B * V1P
    idx1_2d = idx1.reshape(NR1 // 128, 128)
    table1 = h1.reshape(B * N1, F1)
    g1 = _sc_gather(table1, idx1_2d, NR1, F1, c_per_w=40, c_stride=40,
                    dpad=F1)
    g1 = g1.reshape(S, B, V1P, F1)

    W1r = W1.reshape(S, F1, F2)
    h2 = _spiral_mm(g1, W1r, b1.reshape(1, F2), N1, 640, 2, S, F1, F2)
    h3 = _down_mm(D1, h2, 64, 5, F2)

    h3f = h3.reshape(B, N2 * F2)
    return _fc(h3f, Wfc, bfc.reshape(1, LAT), 16128, 5, LAT)
